# baseline (speedup 1.0000x reference)
import functools

import jax
import jax.numpy as jnp
from jax import lax
from jax.experimental import pallas as pl
from jax.experimental.pallas import tpu as pltpu
from jax.experimental.pallas import tpu_sc as plsc

V = 1000000
D = 64
B = 425984
NC = 2
NS = 16
NW = NC * NS
BPW = B // NW
TCOLS = B // 128
TPW = TCOLS // NW
CHUNK = 512
CT = CHUNK // 128
NCHUNK = BPW // CHUNK

_mesh = plsc.VectorSubcoreMesh(core_axis_name="c", subcore_axis_name="s")


@functools.partial(
    pl.kernel,
    mesh=_mesh,
    out_type=jax.ShapeDtypeStruct((8 * TCOLS, 8, 128), jnp.float32),
    scratch_types=[
        pltpu.VMEM((2, CHUNK), jnp.int32),
        pltpu.VMEM((2, CHUNK, D), jnp.float32),
        pltpu.VMEM((8 * CT, 8, 128), jnp.float32),
        pltpu.SemaphoreType.DMA,
        pltpu.SemaphoreType.DMA,
    ],
    compiler_params=pltpu.CompilerParams(
        use_tc_tiling_on_sc=False, needs_layout_passes=False
    ),
)
def _gather(x_hbm, y_hbm, out_hbm, idx_v, rows_v, tiles_v, sem_g, sem_o):
    wid = lax.axis_index("s") * NC + lax.axis_index("c")
    base = wid * BPW
    tbase = wid * TPW

    lanes = lax.iota(jnp.int32, 16)
    zeros = lanes * 0

    pltpu.sync_copy(y_hbm.at[pl.ds(base, CHUNK)], idx_v.at[0])
    pltpu.async_copy(x_hbm.at[idx_v.at[0]], rows_v.at[0], sem_g)
    pltpu.sync_copy(y_hbm.at[pl.ds(base + CHUNK, CHUNK)], idx_v.at[1])

    def chunk_body(ci, _):
        p = ci & 1
        q = 1 - p

        @pl.when(ci + 1 < NCHUNK)
        def _():
            pltpu.async_copy(x_hbm.at[idx_v.at[q]], rows_v.at[q], sem_g)

        pltpu.make_async_copy(
            x_hbm.at[idx_v.at[p]], rows_v.at[p], sem_g).wait()

        @pl.when(ci + 2 < NCHUNK)
        def _():
            pltpu.sync_copy(
                y_hbm.at[pl.ds(base + (ci + 2) * CHUNK, CHUNK)],
                idx_v.at[p])

        rows = rows_v.at[p]

        def group_body(g, _):
            c4 = g >> 3
            mc = g & 7
            rowvec = lanes + 16 * g

            def fbody(t, carries):
                f0 = 4 * t
                new = []
                for u in range(4):
                    f = f0 + u
                    k = f & 7
                    r = (f >> 3) * CT + c4
                    val = plsc.load_gather(rows, [rowvec, carries[u]])
                    tiles_v[r, k, pl.ds(16 * mc, 16)] = val
                    new.append(carries[u] + 4)
                return tuple(new)

            lax.fori_loop(0, 16, fbody,
                          (zeros, zeros + 1, zeros + 2, zeros + 3))
            return 0

        lax.fori_loop(0, 8 * CT, group_body, 0)

        for tr in range(8):
            pltpu.async_copy(
                tiles_v.at[pl.ds(tr * CT, CT)],
                out_hbm.at[pl.ds(tr * TCOLS + tbase + ci * CT, CT)],
                sem_o)
        for tr in range(8):
            pltpu.make_async_copy(
                tiles_v.at[pl.ds(tr * CT, CT)],
                out_hbm.at[pl.ds(tr * TCOLS + tbase + ci * CT, CT)],
                sem_o).wait()
        return 0

    lax.fori_loop(0, NCHUNK, chunk_body, 0)


def kernel(x, y):
    out3 = _gather(x, y)
    return (
        out3.reshape(8, TCOLS, 8, 128).transpose(1, 3, 0, 2).reshape(B, D)
    )


# speedup vs baseline: 1.3855x; 1.3855x over previous
import functools

import jax
import jax.numpy as jnp
from jax import lax
from jax.experimental import pallas as pl
from jax.experimental.pallas import tpu as pltpu
from jax.experimental.pallas import tpu_sc as plsc

V = 1000000
D = 64
B = 425984
NC = 2
NS = 16
NW = NC * NS
BPW = B // NW
CHUNK = 512
NCHUNK = BPW // CHUNK
NBUF = 3
NIDX = 4
LAG = 2

_mesh = plsc.VectorSubcoreMesh(core_axis_name="c", subcore_axis_name="s")


@functools.partial(
    pl.kernel,
    mesh=_mesh,
    out_type=jax.ShapeDtypeStruct((B, D), jnp.float32),
    scratch_types=[
        pltpu.VMEM((NIDX, CHUNK), jnp.int32),
        pltpu.VMEM((NBUF, CHUNK, D), jnp.float32),
        [pltpu.SemaphoreType.DMA] * NIDX,
        [pltpu.SemaphoreType.DMA] * NBUF,
        [pltpu.SemaphoreType.DMA] * NBUF,
    ],
    compiler_params=pltpu.CompilerParams(use_tc_tiling_on_sc=False),
)
def _gather(x_hbm, y_hbm, out_hbm, idx_v, rows_v, sem_i, sem_g, sem_o):
    wid = lax.axis_index("s") * NC + lax.axis_index("c")
    base = wid * BPW

    idx_cp = [None] * NCHUNK
    g_cp = [None] * NCHUNK
    out_cp = [None] * NCHUNK

    def start_idx(i):
        idx_cp[i] = pltpu.async_copy(
            y_hbm.at[pl.ds(base + i * CHUNK, CHUNK)], idx_v.at[i % NIDX],
            sem_i[i % NIDX])

    for i in range(NIDX):
        start_idx(i)

    for i in range(NCHUNK + LAG):
        if i < NCHUNK:
            b = i % NBUF
            if i >= NBUF:
                out_cp[i - NBUF].wait()
            idx_cp[i].wait()
            g_cp[i] = pltpu.async_copy(
                x_hbm.at[idx_v.at[i % NIDX]], rows_v.at[b], sem_g[b])
        j = i - LAG
        if 0 <= j < NCHUNK:
            g_cp[j].wait()
            out_cp[j] = pltpu.async_copy(
                rows_v.at[j % NBUF],
                out_hbm.at[pl.ds(base + j * CHUNK, CHUNK)],
                sem_o[j % NBUF])
            if j + NIDX < NCHUNK:
                start_idx(j + NIDX)


def kernel(x, y):
    return _gather(x, y)
